# Initial kernel scaffold; baseline (speedup 1.0000x reference)
import jax
import jax.numpy as jnp
from jax.experimental import pallas as pl


def kernel(p, x, o, n, idx, ppf_r, down_idx, xpp, r, W_proj, b_proj, W_conv, We1, ge_g, ge_b, We2, W1, ln1_g, ln1_b, W2, ln2_g, ln2_b):
    raise NotImplementedError("write your pallas kernel here")


# speedup vs baseline: 4.6259x; 4.6259x over previous
import functools

import jax
import jax.numpy as jnp
from jax import lax
from jax.experimental import pallas as pl
from jax.experimental.pallas import tpu as pltpu, tpu_sc as plsc

N = 10000
C = 128
NS = 16
C4 = C // 4

NC = 2
NSUB = 16
NW = NC * NSUB
NPAD = 10240
NODES_PER_TILE = NPAD // NW
ROWS_PER_TILE = NODES_PER_TILE * NS
GCH = 128
NCH = ROWS_PER_TILE // GCH

_mesh = plsc.VectorSubcoreMesh(core_axis_name="c", subcore_axis_name="s")
_sc_params = pltpu.CompilerParams(needs_layout_passes=False)


def _worker_id():
    return lax.axis_index("s") * NC + lax.axis_index("c")


def _xc_body(x_ref, w_ref, o_ref):
    o_ref[...] = jnp.dot(x_ref[...], w_ref[...], preferred_element_type=jnp.float32)


def _xc_call(x_pad, wbig_t):
    return pl.pallas_call(
        _xc_body,
        grid=(NPAD // 256,),
        in_specs=[
            pl.BlockSpec((256, C), lambda i: (i, 0)),
            pl.BlockSpec((C, C), lambda i: (0, 0)),
        ],
        out_specs=pl.BlockSpec((256, C), lambda i: (i, 0)),
        out_shape=jax.ShapeDtypeStruct((NPAD, C), jnp.float32),
    )(x_pad, wbig_t)


@functools.partial(
    pl.kernel,
    mesh=_mesh,
    out_type=jax.ShapeDtypeStruct((6, NPAD * NS), jnp.float32),
    scratch_types=[
        pltpu.VMEM((NPAD * 3,), jnp.float32),
        pltpu.VMEM((NPAD * 3,), jnp.float32),
        pltpu.VMEM((ROWS_PER_TILE,), jnp.int32),
        pltpu.VMEM((6 * ROWS_PER_TILE,), jnp.float32),
    ],
    compiler_params=_sc_params,
)
def _geo_kernel(p_hbm, n_hbm, idx_hbm, geo_hbm, p_v, n_v, idx_v, geo_v):
    wid = _worker_id()
    base = wid * NODES_PER_TILE
    pltpu.sync_copy(p_hbm, p_v)
    pltpu.sync_copy(n_hbm, n_v)
    pltpu.sync_copy(idx_hbm.at[pl.ds(base * NS, ROWS_PER_TILE)], idx_v)

    def body(li, carry):
        gi = base + li
        gvec = jnp.full((NS,), 3 * gi, jnp.int32)
        nbr = idx_v[pl.ds(li * NS, NS)] * 3
        px = plsc.load_gather(p_v, [nbr])
        py = plsc.load_gather(p_v, [nbr + 1])
        pz = plsc.load_gather(p_v, [nbr + 2])
        ngx = plsc.load_gather(n_v, [nbr])
        ngy = plsc.load_gather(n_v, [nbr + 1])
        ngz = plsc.load_gather(n_v, [nbr + 2])
        pix = plsc.load_gather(p_v, [gvec])
        piy = plsc.load_gather(p_v, [gvec + 1])
        piz = plsc.load_gather(p_v, [gvec + 2])
        nix = plsc.load_gather(n_v, [gvec])
        niy = plsc.load_gather(n_v, [gvec + 1])
        niz = plsc.load_gather(n_v, [gvec + 2])
        dx = px - pix
        dy = py - piy
        dz = pz - piz
        off = li * NS
        geo_v[pl.ds(off, NS)] = dx * dx + dy * dy + dz * dz
        geo_v[pl.ds(off + ROWS_PER_TILE, NS)] = nix * dx + niy * dy + niz * dz
        geo_v[pl.ds(off + 2 * ROWS_PER_TILE, NS)] = ngx * dx + ngy * dy + ngz * dz
        geo_v[pl.ds(off + 3 * ROWS_PER_TILE, NS)] = (
            nix * ngx + niy * ngy + niz * ngz)
        geo_v[pl.ds(off + 4 * ROWS_PER_TILE, NS)] = (
            ngx * ngx + ngy * ngy + ngz * ngz)
        geo_v[pl.ds(off + 5 * ROWS_PER_TILE, NS)] = (
            nix * nix + niy * niy + niz * niz)
        return carry

    lax.fori_loop(0, NODES_PER_TILE, body, 0)
    for f in range(6):
        pltpu.sync_copy(
            geo_v.at[pl.ds(f * ROWS_PER_TILE, ROWS_PER_TILE)],
            geo_hbm.at[f, pl.ds(base * NS, ROWS_PER_TILE)])


@functools.partial(
    pl.kernel,
    mesh=_mesh,
    out_type=jax.ShapeDtypeStruct((NPAD * NS, C), jnp.float32),
    scratch_types=[
        pltpu.VMEM((NCH, GCH), jnp.int32),
        pltpu.VMEM((GCH, C), jnp.float32),
        pltpu.VMEM((GCH, C), jnp.float32),
        pltpu.SemaphoreType.DMA,
        pltpu.SemaphoreType.DMA,
    ],
    compiler_params=_sc_params,
)
def _gather_kernel(xc_hbm, idx_hbm, out_hbm, idx_v, buf_a, buf_b, sem_a, sem_b):
    wid = _worker_id()
    row_base = wid * ROWS_PER_TILE
    pltpu.sync_copy(idx_hbm.at[pl.ds(wid * NCH, NCH)], idx_v)
    bufs = (buf_a, buf_b)
    sems = (sem_a, sem_b)
    copies = [None, None]
    copies[0] = pltpu.async_copy(xc_hbm.at[idx_v.at[0]], bufs[0], sems[0])
    for c in range(NCH):
        cur = c % 2
        nxt = (c + 1) % 2
        if c + 1 < NCH:
            copies[nxt] = pltpu.async_copy(
                xc_hbm.at[idx_v.at[c + 1]], bufs[nxt], sems[nxt])
        copies[cur].wait()
        pltpu.sync_copy(bufs[cur], out_hbm.at[pl.ds(row_base + c * GCH, GCH)])


BLK = 256
RWS = BLK * NS


def _dense_body(dd_ref, n1d_ref, ngd_ref, n1ng_ref, ngng_ref, nn_ref,
                xg_ref, x_ref, we1_ref, geg_ref, geb_ref, we2pt_ref,
                w1t_ref, l1g_ref, l1b_ref, w2t_ref, l2g_ref, l2b_ref,
                wpt_ref, bp_ref, o_ref):
    eps = 1e-8
    dd = dd_ref[...]
    n1d = n1d_ref[...]
    ngd = ngd_ref[...]
    n1ng = n1ng_ref[...]
    ngng = ngng_ref[...]
    nn = nn_ref[...]

    def ang(num, aa, bb):
        cos = num / (jnp.sqrt(aa * bb) + eps)
        cos = jnp.clip(cos, -1.0 + 1e-6, 1.0 - 1e-6)
        return jnp.arctan2(jnp.sqrt((1.0 + cos) * (1.0 - cos)), cos)

    a1 = ang(n1d, nn, dd)
    a2 = ang(ngd, ngng, dd)
    a3 = ang(n1ng, nn, ngng)
    len_d = jnp.sqrt(dd + 1e-8)

    pe = (a1[:, :, None] * we1_ref[0][None, None, :]
          + a2[:, :, None] * we1_ref[1][None, None, :]
          + a3[:, :, None] * we1_ref[2][None, None, :]
          + len_d[:, :, None] * we1_ref[3][None, None, :])
    m = jnp.mean(pe, axis=-1, keepdims=True)
    v = jnp.mean((pe - m) ** 2, axis=-1, keepdims=True)
    h = (pe - m) * lax.rsqrt(v + 1e-5) * geg_ref[0][None, None, :] \
        + geb_ref[0][None, None, :]
    h = jnp.where(h > 0, h, 0.1 * h)
    h = h.reshape(RWS, C)
    pec = jnp.dot(h, we2pt_ref[...], preferred_element_type=jnp.float32)
    hc = (xg_ref[...] + pec).reshape(BLK, NS, C)
    hm = jnp.max(hc, axis=1)

    def ln(t, g, b):
        mm = jnp.mean(t, axis=-1, keepdims=True)
        vv = jnp.mean((t - mm) ** 2, axis=-1, keepdims=True)
        return (t - mm) * lax.rsqrt(vv + 1e-5) * g + b

    t = ln(jnp.dot(hm, w1t_ref[...], preferred_element_type=jnp.float32),
           l1g_ref[...], l1b_ref[...])
    t = jnp.where(t > 0, t, 0.1 * t)
    t = ln(jnp.dot(t, w2t_ref[...], preferred_element_type=jnp.float32),
           l2g_ref[...], l2b_ref[...])
    t = jnp.maximum(t + x_ref[...], 0.0)
    o_ref[...] = jnp.dot(t, wpt_ref[...], preferred_element_type=jnp.float32) \
        + bp_ref[...]


def _dense_call(geos, xg, x_pad, we1_eff, ge_g, ge_b, we2pt, w1t, l1g, l1b,
                w2t, l2g, l2b, wpt, bp):
    full = lambda shape: pl.BlockSpec(shape, lambda i: tuple(0 for _ in shape))
    geo_spec = pl.BlockSpec((BLK, NS), lambda i: (i, 0))
    return pl.pallas_call(
        _dense_body,
        grid=(NPAD // BLK,),
        in_specs=[
            geo_spec, geo_spec, geo_spec, geo_spec, geo_spec, geo_spec,
            pl.BlockSpec((RWS, C), lambda i: (i, 0)),
            pl.BlockSpec((BLK, C), lambda i: (i, 0)),
            full((8, C)),
            full((1, C)),
            full((1, C)),
            full((C, C)),
            full((C, 4 * C)),
            full((1, 4 * C)),
            full((1, 4 * C)),
            full((4 * C, C)),
            full((1, C)),
            full((1, C)),
            full((C, C)),
            full((1, C)),
        ],
        out_specs=pl.BlockSpec((BLK, C), lambda i: (i, 0)),
        out_shape=jax.ShapeDtypeStruct((NPAD, C), jnp.float32),
    )(*geos, xg, x_pad, we1_eff, ge_g, ge_b, we2pt, w1t, l1g, l1b,
      w2t, l2g, l2b, wpt, bp)


def kernel(p, x, o, n, idx, ppf_r, down_idx, xpp, r, W_proj, b_proj, W_conv,
           We1, ge_g, ge_b, We2, W1, ln1_g, ln1_b, W2, ln2_g, ln2_b):
    f32 = jnp.float32
    pad_n = NPAD - N
    p_pad = jnp.pad(p.astype(f32), ((0, pad_n), (0, 0)))
    n_pad = jnp.pad(n.astype(f32), ((0, pad_n), (0, 0)))
    x_pad = jnp.pad(x.astype(f32), ((0, pad_n), (0, 0)))
    g16 = jnp.pad(idx[:, :NS].astype(jnp.int32), ((0, pad_n), (0, 0)))
    gflat2d = g16.reshape(NPAD * NS // GCH, GCH)

    wbig = jnp.zeros((C, C), f32).at[:C4, :C4].set(W_conv)
    wbig = wbig.at[C4:, C4:].set(jnp.eye(C - C4, dtype=f32))
    we2p_t = (wbig @ We2).T
    w3_eff = We1[:, 3] + We1[:, 4] / (r[0] + 1e-8)
    we1_eff = jnp.zeros((8, C), f32)
    we1_eff = we1_eff.at[0].set(We1[:, 0]).at[1].set(We1[:, 1])
    we1_eff = we1_eff.at[2].set(We1[:, 2]).at[3].set(w3_eff)

    xc = _xc_call(x_pad, wbig.T)
    geo = _geo_kernel(p_pad.reshape(-1), n_pad.reshape(-1), g16.reshape(-1))
    xg = _gather_kernel(xc, gflat2d)
    geo3 = geo.reshape(6, NPAD, NS)
    geos = [geo3[f] for f in range(6)]
    out = _dense_call(
        geos, xg, x_pad, we1_eff,
        ge_g.reshape(1, C), ge_b.reshape(1, C), we2p_t,
        W1.T, ln1_g.reshape(1, 4 * C), ln1_b.reshape(1, 4 * C),
        W2.T, ln2_g.reshape(1, C), ln2_b.reshape(1, C),
        W_proj.T, b_proj.reshape(1, C))
    return out[:N]
